# baseline (speedup 1.0000x reference)
import jax
import jax.numpy as jnp
from jax.experimental import pallas as pl
from jax.experimental.pallas import tpu as pltpu

D = 512
H = 16
HD = H * D
B = 16384

KCH = 512
N_KCH = HD // KCH
WBUF = 6
CH = 1024
N_CH = B // CH
NBUF = 10


def _kp_copy(i, kp_hbm, kp_v, ksem):
    return pltpu.make_async_copy(
        kp_hbm.at[pl.ds(i * KCH, KCH), :], kp_v.at[i % WBUF], ksem.at[i % WBUF])


def _mem_copy(i, mem_hbm, mem_v, msem):
    return pltpu.make_async_copy(
        mem_hbm.at[pl.ds(i * KCH, KCH), :], mem_v.at[i % WBUF], msem.at[i % WBUF])


def _z_copy(c, z_hbm, z_v, zsem):
    return pltpu.make_async_copy(
        z_hbm.at[pl.ds(c * CH, CH), :], z_v.at[c % NBUF], zsem.at[c % NBUF])


def _out_copy(c, out_v, out_hbm, osem):
    return pltpu.make_async_copy(
        out_v.at[c % NBUF], out_hbm.at[pl.ds(c * CH, CH), :], osem.at[c % NBUF])


def _fused_kernel(kp_hbm, mem_hbm, wout_hbm, z_hbm, out_hbm,
                  kp_v, mem_v, wout_v, z_v, out_v, acc_v, e_v,
                  wsem, ksem, msem, zsem, osem):
    for i in range(WBUF):
        _kp_copy(i, kp_hbm, kp_v, ksem).start()
        _mem_copy(i, mem_hbm, mem_v, msem).start()
    wout_copy = pltpu.make_async_copy(wout_hbm, wout_v, wsem)
    wout_copy.start()

    for i in range(N_KCH):
        _kp_copy(i, kp_hbm, kp_v, ksem).wait()
        _mem_copy(i, mem_hbm, mem_v, msem).wait()
        if i + WBUF < N_KCH:
            _kp_copy(i + WBUF, kp_hbm, kp_v, ksem).start()
            _mem_copy(i + WBUF, mem_hbm, mem_v, msem).start()
        if i < NBUF:
            _z_copy(i, z_hbm, z_v, zsem).start()
        part = jax.lax.dot_general(
            kp_v[i % WBUF], mem_v[i % WBUF],
            (((0,), (0,)), ((), ())),
            preferred_element_type=jnp.float32,
        )
        if i == 0:
            acc_v[...] = part
        else:
            acc_v[...] += part
    wout_copy.wait()
    e_v[...] = jax.lax.dot_general(
        acc_v[...] * (1.0 / H), wout_v[...],
        (((1,), (1,)), ((), ())),
        preferred_element_type=jnp.float32,
    )

    for c in range(N_CH):
        _z_copy(c, z_hbm, z_v, zsem).wait()
        if c >= NBUF:
            _out_copy(c - NBUF, out_v, out_hbm, osem).wait()
        out_v[c % NBUF] = jnp.dot(
            z_v[c % NBUF], e_v[...], preferred_element_type=jnp.float32
        )
        _out_copy(c, out_v, out_hbm, osem).start()
        if c + NBUF < N_CH:
            _z_copy(c + NBUF, z_hbm, z_v, zsem).start()
    for c in range(N_CH - NBUF, N_CH):
        _out_copy(c, out_v, out_hbm, osem).wait()


@jax.jit
def kernel(z_query, key_proj_w, out_proj_w, memory):
    mem_2d = memory.reshape(HD, D)
    out = pl.pallas_call(
        _fused_kernel,
        in_specs=[pl.BlockSpec(memory_space=pltpu.MemorySpace.HBM)] * 4,
        out_specs=pl.BlockSpec(memory_space=pltpu.MemorySpace.HBM),
        out_shape=jax.ShapeDtypeStruct((B, D), jnp.float32),
        scratch_shapes=[
            pltpu.VMEM((WBUF, KCH, D), jnp.float32),
            pltpu.VMEM((WBUF, KCH, D), jnp.float32),
            pltpu.VMEM((D, D), jnp.float32),
            pltpu.VMEM((NBUF, CH, D), jnp.float32),
            pltpu.VMEM((NBUF, CH, D), jnp.float32),
            pltpu.VMEM((D, D), jnp.float32),
            pltpu.VMEM((D, D), jnp.float32),
            pltpu.SemaphoreType.DMA,
            pltpu.SemaphoreType.DMA((WBUF,)),
            pltpu.SemaphoreType.DMA((WBUF,)),
            pltpu.SemaphoreType.DMA((NBUF,)),
            pltpu.SemaphoreType.DMA((NBUF,)),
        ],
    )(key_proj_w, mem_2d, out_proj_w, z_query)
    return out


# speedup vs baseline: 1.0815x; 1.0815x over previous
import jax
import jax.numpy as jnp
from jax.experimental import pallas as pl
from jax.experimental.pallas import tpu as pltpu

D = 512
H = 16
HD = H * D
B = 16384

KCH = 1024
N_KCH = HD // KCH
WBUF = 3
CH = 1024
N_CH = B // CH
NBUF = 10


def _kp_copy(i, kp_hbm, kp_v, ksem):
    return pltpu.make_async_copy(
        kp_hbm.at[pl.ds(i * KCH, KCH), :], kp_v.at[i % WBUF], ksem.at[i % WBUF])


def _mem_copy(i, mem_hbm, mem_v, msem):
    return pltpu.make_async_copy(
        mem_hbm.at[pl.ds(i * KCH, KCH), :], mem_v.at[i % WBUF], msem.at[i % WBUF])


def _z_copy(c, z_hbm, z_v, zsem):
    return pltpu.make_async_copy(
        z_hbm.at[pl.ds(c * CH, CH), :], z_v.at[c % NBUF], zsem.at[c % NBUF])


def _out_copy(c, out_v, out_hbm, osem):
    return pltpu.make_async_copy(
        out_v.at[c % NBUF], out_hbm.at[pl.ds(c * CH, CH), :], osem.at[c % NBUF])


def _fused_kernel(kp_hbm, mem_hbm, wout_hbm, z_hbm, out_hbm,
                  kp_v, mem_v, wout_v, z_v, out_v, acc_v, e_v,
                  wsem, ksem, msem, zsem, osem):
    for i in range(WBUF):
        _kp_copy(i, kp_hbm, kp_v, ksem).start()
        _mem_copy(i, mem_hbm, mem_v, msem).start()
    wout_copy = pltpu.make_async_copy(wout_hbm, wout_v, wsem)
    wout_copy.start()

    for i in range(N_KCH):
        _kp_copy(i, kp_hbm, kp_v, ksem).wait()
        _mem_copy(i, mem_hbm, mem_v, msem).wait()
        if i + WBUF < N_KCH:
            _kp_copy(i + WBUF, kp_hbm, kp_v, ksem).start()
            _mem_copy(i + WBUF, mem_hbm, mem_v, msem).start()
        _z_copy(i, z_hbm, z_v, zsem).start()
        part = jax.lax.dot_general(
            kp_v[i % WBUF], mem_v[i % WBUF],
            (((0,), (0,)), ((), ())),
            preferred_element_type=jnp.float32,
        )
        if i == 0:
            acc_v[...] = part
        else:
            acc_v[...] += part
    for c in range(N_KCH, NBUF):
        _z_copy(c, z_hbm, z_v, zsem).start()
    wout_copy.wait()
    e_v[...] = jax.lax.dot_general(
        acc_v[...] * (1.0 / H), wout_v[...],
        (((1,), (1,)), ((), ())),
        preferred_element_type=jnp.float32,
    )

    for c in range(N_CH):
        _z_copy(c, z_hbm, z_v, zsem).wait()
        if c >= NBUF:
            _out_copy(c - NBUF, out_v, out_hbm, osem).wait()
        out_v[c % NBUF] = jnp.dot(
            z_v[c % NBUF], e_v[...], preferred_element_type=jnp.float32
        )
        _out_copy(c, out_v, out_hbm, osem).start()
        if c + NBUF < N_CH:
            _z_copy(c + NBUF, z_hbm, z_v, zsem).start()
    for c in range(N_CH - NBUF, N_CH):
        _out_copy(c, out_v, out_hbm, osem).wait()


@jax.jit
def kernel(z_query, key_proj_w, out_proj_w, memory):
    mem_2d = memory.reshape(HD, D)
    out = pl.pallas_call(
        _fused_kernel,
        in_specs=[pl.BlockSpec(memory_space=pltpu.MemorySpace.HBM)] * 4,
        out_specs=pl.BlockSpec(memory_space=pltpu.MemorySpace.HBM),
        out_shape=jax.ShapeDtypeStruct((B, D), jnp.float32),
        scratch_shapes=[
            pltpu.VMEM((WBUF, KCH, D), jnp.float32),
            pltpu.VMEM((WBUF, KCH, D), jnp.float32),
            pltpu.VMEM((D, D), jnp.float32),
            pltpu.VMEM((NBUF, CH, D), jnp.float32),
            pltpu.VMEM((NBUF, CH, D), jnp.float32),
            pltpu.VMEM((D, D), jnp.float32),
            pltpu.VMEM((D, D), jnp.float32),
            pltpu.SemaphoreType.DMA,
            pltpu.SemaphoreType.DMA((WBUF,)),
            pltpu.SemaphoreType.DMA((WBUF,)),
            pltpu.SemaphoreType.DMA((NBUF,)),
            pltpu.SemaphoreType.DMA((NBUF,)),
        ],
    )(key_proj_w, mem_2d, out_proj_w, z_query)
    return out
